# Initial kernel scaffold; baseline (speedup 1.0000x reference)
import jax
import jax.numpy as jnp
from jax.experimental import pallas as pl


def kernel(x, edge_index, eps, W1, b1, W2, b2):
    raise NotImplementedError("write your pallas kernel here")


# speedup vs baseline: 10.9681x; 10.9681x over previous
import functools

import jax
import jax.numpy as jnp
from jax import lax
from jax.experimental import pallas as pl
from jax.experimental.pallas import tpu as pltpu
from jax.experimental.pallas import tpu_sc as plsc

_N, _E, _C, _H = 10000, 320000, 128, 16
_NC, _NS = 2, 16
_NW = _NC * _NS
_CH = 80
_RPW = _E // (_NW * _CH)
_SEG = 624
_BM = 2000


def _mm1_body(x_ref, w_ref, y_ref):
    y_ref[...] = jnp.dot(x_ref[...], w_ref[...],
                         preferred_element_type=jnp.float32)


def _tail_body(eps_ref, y_ref, a0_ref, a1_ref, b1_ref, w2_ref, b2_ref, o_ref):
    h = (1.0 + eps_ref[0, 0]) * y_ref[...] + a0_ref[...] + a1_ref[...]
    h = jnp.maximum(h + b1_ref[...], 0.0)
    o = jnp.dot(h, w2_ref[...], preferred_element_type=jnp.float32)
    o_ref[...] = jnp.maximum(o + b2_ref[...], 0.0)


@functools.partial(
    pl.kernel,
    out_type=jax.ShapeDtypeStruct((_NC, _N, _H), jnp.float32),
    mesh=plsc.VectorSubcoreMesh(core_axis_name="c", subcore_axis_name="s"),
    compiler_params=pltpu.CompilerParams(use_tc_tiling_on_sc=False),
    scratch_types=[
        pltpu.VMEM((_RPW, _CH), jnp.int32),
        pltpu.VMEM((_RPW, _CH), jnp.int32),
        pltpu.VMEM((_CH, _H), jnp.float32),
        pltpu.VMEM((_SEG + _NS, _H), jnp.float32),
        pltpu.VMEM_SHARED((_N, _H), jnp.float32),
        pltpu.SemaphoreType.DMA,
    ],
)
def _sc_aggregate(y_hbm, src_hbm, dst_hbm, out_hbm,
                  src_v, dst_v, rows_v, zero_v, aggr_sh, sem):
    c = lax.axis_index("c")
    s = lax.axis_index("s")
    wid = s * _NC + c
    last_seg = _N - (_NS - 1) * _SEG
    last_off = (_NS - 1) * _SEG

    def _zrow(i, carry):
        zero_v[i, :] = jnp.zeros((_H,), jnp.float32)
        return carry

    lax.fori_loop(0, _SEG + _NS, _zrow, 0)

    @pl.when(s < _NS - 1)
    def _():
        pltpu.sync_copy(zero_v.at[pl.ds(0, _SEG)],
                        aggr_sh.at[pl.ds(s * _SEG, _SEG)])

    @pl.when(s == _NS - 1)
    def _():
        pltpu.sync_copy(zero_v, aggr_sh.at[pl.ds(last_off, last_seg)])

    pltpu.sync_copy(src_hbm.at[wid], src_v)
    pltpu.sync_copy(dst_hbm.at[wid], dst_v)
    plsc.subcore_barrier()

    def _edge_chunk(j, carry):
        pltpu.async_copy(y_hbm.at[src_v.at[j]], rows_v, sem).wait()
        pltpu.sync_copy(rows_v, aggr_sh.at[dst_v.at[j]], add=True)
        return carry

    lax.fori_loop(0, _RPW, _edge_chunk, 0)
    plsc.subcore_barrier()

    @pl.when(s < _NS - 1)
    def _():
        pltpu.sync_copy(aggr_sh.at[pl.ds(s * _SEG, _SEG)],
                        out_hbm.at[c, pl.ds(s * _SEG, _SEG)])

    @pl.when(s == _NS - 1)
    def _():
        pltpu.sync_copy(aggr_sh.at[pl.ds(last_off, last_seg)],
                        out_hbm.at[c, pl.ds(last_off, last_seg)])


def kernel(x, edge_index, eps, W1, b1, W2, b2):
    src = edge_index[0].reshape(_NW, _RPW, _CH)
    dst = edge_index[1].reshape(_NW, _RPW, _CH)

    y = pl.pallas_call(
        _mm1_body,
        grid=(_N // _BM,),
        in_specs=[
            pl.BlockSpec((_BM, _C), lambda i: (i, 0)),
            pl.BlockSpec((_C, _H), lambda i: (0, 0)),
        ],
        out_specs=pl.BlockSpec((_BM, _H), lambda i: (i, 0)),
        out_shape=jax.ShapeDtypeStruct((_N, _H), jnp.float32),
    )(x, W1)

    parts = _sc_aggregate(y, src, dst)

    out = pl.pallas_call(
        _tail_body,
        grid=(_N // _BM,),
        in_specs=[
            pl.BlockSpec(memory_space=pltpu.SMEM),
            pl.BlockSpec((_BM, _H), lambda i: (i, 0)),
            pl.BlockSpec((_BM, _H), lambda i: (i, 0)),
            pl.BlockSpec((_BM, _H), lambda i: (i, 0)),
            pl.BlockSpec((1, _H), lambda i: (0, 0)),
            pl.BlockSpec((_H, _H), lambda i: (0, 0)),
            pl.BlockSpec((1, _H), lambda i: (0, 0)),
        ],
        out_specs=pl.BlockSpec((_BM, _H), lambda i: (i, 0)),
        out_shape=jax.ShapeDtypeStruct((_N, _H), jnp.float32),
    )(jnp.reshape(eps, (1, 1)), y, parts[0], parts[1],
      jnp.reshape(b1, (1, _H)), W2, jnp.reshape(b2, (1, _H)))

    return out


# speedup vs baseline: 19.1839x; 1.7491x over previous
import functools

import jax
import jax.numpy as jnp
from jax import lax
from jax.experimental import pallas as pl
from jax.experimental.pallas import tpu as pltpu
from jax.experimental.pallas import tpu_sc as plsc

_N, _E, _C, _H = 10000, 320000, 128, 16
_NC, _NS = 2, 16
_NW = _NC * _NS
_CH = 80
_RPW = _E // (_NW * _CH)
_SEG = 624
_NB = 5
_BM = 2000


def _mm1_body(x_ref, w_ref, y_ref):
    y_ref[...] = jnp.dot(x_ref[...], w_ref[...],
                         preferred_element_type=jnp.float32)


def _tail_body(eps_ref, y_ref, a0_ref, a1_ref, b1_ref, w2_ref, b2_ref, o_ref):
    h = (1.0 + eps_ref[0, 0]) * y_ref[...] + a0_ref[...] + a1_ref[...]
    h = jnp.maximum(h + b1_ref[...], 0.0)
    o = jnp.dot(h, w2_ref[...], preferred_element_type=jnp.float32)
    o_ref[...] = jnp.maximum(o + b2_ref[...], 0.0)


@functools.partial(
    pl.kernel,
    out_type=jax.ShapeDtypeStruct((_NC, _N, _H), jnp.float32),
    mesh=plsc.VectorSubcoreMesh(core_axis_name="c", subcore_axis_name="s"),
    compiler_params=pltpu.CompilerParams(use_tc_tiling_on_sc=False),
    scratch_types=[
        pltpu.VMEM((_RPW, _CH), jnp.int32),
        pltpu.VMEM((_RPW, _CH), jnp.int32),
        pltpu.VMEM((_NB, _CH, _H), jnp.float32),
        pltpu.VMEM((_SEG + _NS, _H), jnp.float32),
        pltpu.VMEM_SHARED((_N, _H), jnp.float32),
        [pltpu.SemaphoreType.DMA] * _NB,
    ],
)
def _sc_aggregate(y_hbm, src_hbm, dst_hbm, out_hbm,
                  src_v, dst_v, rows_v, zero_v, aggr_sh, sems):
    c = lax.axis_index("c")
    s = lax.axis_index("s")
    wid = s * _NC + c
    last_seg = _N - (_NS - 1) * _SEG
    last_off = (_NS - 1) * _SEG

    def _zrow(i, carry):
        zero_v[i, :] = jnp.zeros((_H,), jnp.float32)
        return carry

    lax.fori_loop(0, _SEG + _NS, _zrow, 0)

    @pl.when(s < _NS - 1)
    def _():
        pltpu.sync_copy(zero_v.at[pl.ds(0, _SEG)],
                        aggr_sh.at[pl.ds(s * _SEG, _SEG)])

    @pl.when(s == _NS - 1)
    def _():
        pltpu.sync_copy(zero_v, aggr_sh.at[pl.ds(last_off, last_seg)])

    pltpu.sync_copy(src_hbm.at[wid], src_v)
    pltpu.sync_copy(dst_hbm.at[wid], dst_v)
    plsc.subcore_barrier()

    for b in range(_NB):
        pltpu.async_copy(y_hbm.at[src_v.at[b]], rows_v.at[b], sems[b])

    def _edge_group(g, carry):
        j0 = g * _NB
        for b in range(_NB):
            j = j0 + b
            pltpu.make_async_copy(y_hbm.at[src_v.at[b]],
                                  rows_v.at[b], sems[b]).wait()
            pltpu.sync_copy(rows_v.at[b], aggr_sh.at[dst_v.at[j]], add=True)

            @pl.when(j + _NB < _RPW)
            def _():
                pltpu.async_copy(y_hbm.at[src_v.at[j + _NB]],
                                 rows_v.at[b], sems[b])
        return carry

    lax.fori_loop(0, _RPW // _NB, _edge_group, 0)
    plsc.subcore_barrier()

    @pl.when(s < _NS - 1)
    def _():
        pltpu.sync_copy(aggr_sh.at[pl.ds(s * _SEG, _SEG)],
                        out_hbm.at[c, pl.ds(s * _SEG, _SEG)])

    @pl.when(s == _NS - 1)
    def _():
        pltpu.sync_copy(aggr_sh.at[pl.ds(last_off, last_seg)],
                        out_hbm.at[c, pl.ds(last_off, last_seg)])


def kernel(x, edge_index, eps, W1, b1, W2, b2):
    src = edge_index[0].reshape(_NW, _RPW, _CH)
    dst = edge_index[1].reshape(_NW, _RPW, _CH)

    y = pl.pallas_call(
        _mm1_body,
        grid=(_N // _BM,),
        in_specs=[
            pl.BlockSpec((_BM, _C), lambda i: (i, 0)),
            pl.BlockSpec((_C, _H), lambda i: (0, 0)),
        ],
        out_specs=pl.BlockSpec((_BM, _H), lambda i: (i, 0)),
        out_shape=jax.ShapeDtypeStruct((_N, _H), jnp.float32),
    )(x, W1)

    parts = _sc_aggregate(y, src, dst)

    out = pl.pallas_call(
        _tail_body,
        grid=(_N // _BM,),
        in_specs=[
            pl.BlockSpec(memory_space=pltpu.SMEM),
            pl.BlockSpec((_BM, _H), lambda i: (i, 0)),
            pl.BlockSpec((_BM, _H), lambda i: (i, 0)),
            pl.BlockSpec((_BM, _H), lambda i: (i, 0)),
            pl.BlockSpec((1, _H), lambda i: (0, 0)),
            pl.BlockSpec((_H, _H), lambda i: (0, 0)),
            pl.BlockSpec((1, _H), lambda i: (0, 0)),
        ],
        out_specs=pl.BlockSpec((_BM, _H), lambda i: (i, 0)),
        out_shape=jax.ShapeDtypeStruct((_N, _H), jnp.float32),
    )(jnp.reshape(eps, (1, 1)), y, parts[0], parts[1],
      jnp.reshape(b1, (1, _H)), W2, jnp.reshape(b2, (1, _H)))

    return out


# speedup vs baseline: 20.3943x; 1.0631x over previous
import functools

import jax
import jax.numpy as jnp
from jax import lax
from jax.experimental import pallas as pl
from jax.experimental.pallas import tpu as pltpu
from jax.experimental.pallas import tpu_sc as plsc

_N, _E, _C, _H = 10000, 320000, 128, 16
_NC, _NS = 2, 16
_NW = _NC * _NS
_CH = 80
_RPW = _E // (_NW * _CH)
_SEG = 624
_NB = 5
_NP = _N // 8


def _mm1_body(x_ref, w_ref, y_ref):
    y_ref[...] = jnp.dot(x_ref[...], w_ref[...],
                         preferred_element_type=jnp.float32)


def _tail_body(eps_ref, y_ref, a0_ref, a1_ref, b1_ref, w2_ref, b2_ref, o_ref):
    h = (1.0 + eps_ref[0, 0]) * y_ref[...] + a0_ref[...] + a1_ref[...]
    h = jnp.maximum(h + b1_ref[...], 0.0)
    o = jnp.dot(h, w2_ref[...], preferred_element_type=jnp.float32)
    o_ref[...] = jnp.maximum(o + b2_ref[...], 0.0)


@functools.partial(
    pl.kernel,
    out_type=jax.ShapeDtypeStruct((_NC, _N, _H), jnp.float32),
    mesh=plsc.VectorSubcoreMesh(core_axis_name="c", subcore_axis_name="s"),
    compiler_params=pltpu.CompilerParams(use_tc_tiling_on_sc=False),
    scratch_types=[
        pltpu.VMEM((_RPW, _CH), jnp.int32),
        pltpu.VMEM((_RPW, _CH), jnp.int32),
        pltpu.VMEM((_NB, _CH, _H), jnp.float32),
        pltpu.VMEM((_SEG + _NS, _H), jnp.float32),
        pltpu.VMEM_SHARED((_N, _H), jnp.float32),
        [pltpu.SemaphoreType.DMA] * _NB,
    ],
)
def _sc_aggregate(y_hbm, src_hbm, dst_hbm, out_hbm,
                  src_v, dst_v, rows_v, zero_v, aggr_sh, sems):
    c = lax.axis_index("c")
    s = lax.axis_index("s")
    wid = s * _NC + c
    last_seg = _N - (_NS - 1) * _SEG
    last_off = (_NS - 1) * _SEG

    def _zrow(i, carry):
        zero_v[i, :] = jnp.zeros((_H,), jnp.float32)
        return carry

    lax.fori_loop(0, _SEG + _NS, _zrow, 0)

    @pl.when(s < _NS - 1)
    def _():
        pltpu.sync_copy(zero_v.at[pl.ds(0, _SEG)],
                        aggr_sh.at[pl.ds(s * _SEG, _SEG)])

    @pl.when(s == _NS - 1)
    def _():
        pltpu.sync_copy(zero_v, aggr_sh.at[pl.ds(last_off, last_seg)])

    pltpu.sync_copy(src_hbm.at[wid], src_v)
    pltpu.sync_copy(dst_hbm.at[wid], dst_v)
    plsc.subcore_barrier()

    for b in range(_NB):
        pltpu.async_copy(y_hbm.at[src_v.at[b]], rows_v.at[b], sems[b])

    def _edge_group(g, carry):
        j0 = g * _NB
        for b in range(_NB):
            j = j0 + b
            pltpu.make_async_copy(y_hbm.at[src_v.at[b]],
                                  rows_v.at[b], sems[b]).wait()
            pltpu.sync_copy(rows_v.at[b], aggr_sh.at[dst_v.at[j]], add=True)

            @pl.when(j + _NB < _RPW)
            def _():
                pltpu.async_copy(y_hbm.at[src_v.at[j + _NB]],
                                 rows_v.at[b], sems[b])
        return carry

    lax.fori_loop(0, _RPW // _NB, _edge_group, 0)
    plsc.subcore_barrier()

    @pl.when(s < _NS - 1)
    def _():
        pltpu.sync_copy(aggr_sh.at[pl.ds(s * _SEG, _SEG)],
                        out_hbm.at[c, pl.ds(s * _SEG, _SEG)])

    @pl.when(s == _NS - 1)
    def _():
        pltpu.sync_copy(aggr_sh.at[pl.ds(last_off, last_seg)],
                        out_hbm.at[c, pl.ds(last_off, last_seg)])


def _blockdiag8(w):
    k, m = w.shape
    return (jnp.eye(8, dtype=w.dtype)[:, None, :, None]
            * w[None, :, None, :]).reshape(8 * k, 8 * m)


def kernel(x, edge_index, eps, W1, b1, W2, b2):
    src = edge_index[0].reshape(_NW, _RPW, _CH)
    dst = edge_index[1].reshape(_NW, _RPW, _CH)

    xp = x.reshape(_NP, 8 * _C)
    w1p = _blockdiag8(W1)
    w2p = _blockdiag8(W2)
    b1p = jnp.tile(b1, 8).reshape(1, 8 * _H)
    b2p = jnp.tile(b2, 8).reshape(1, 8 * _H)

    y128 = pl.pallas_call(
        _mm1_body,
        grid=(1,),
        in_specs=[
            pl.BlockSpec((_NP, 8 * _C), lambda i: (0, 0)),
            pl.BlockSpec((8 * _C, 8 * _H), lambda i: (0, 0)),
        ],
        out_specs=pl.BlockSpec((_NP, 8 * _H), lambda i: (0, 0)),
        out_shape=jax.ShapeDtypeStruct((_NP, 8 * _H), jnp.float32),
    )(xp, w1p)

    parts = _sc_aggregate(y128.reshape(_N, _H), src, dst)
    pp = parts.reshape(_NC, _NP, 8 * _H)

    out128 = pl.pallas_call(
        _tail_body,
        grid=(1,),
        in_specs=[
            pl.BlockSpec(memory_space=pltpu.SMEM),
            pl.BlockSpec((_NP, 8 * _H), lambda i: (0, 0)),
            pl.BlockSpec((_NP, 8 * _H), lambda i: (0, 0)),
            pl.BlockSpec((_NP, 8 * _H), lambda i: (0, 0)),
            pl.BlockSpec((1, 8 * _H), lambda i: (0, 0)),
            pl.BlockSpec((8 * _H, 8 * _H), lambda i: (0, 0)),
            pl.BlockSpec((1, 8 * _H), lambda i: (0, 0)),
        ],
        out_specs=pl.BlockSpec((_NP, 8 * _H), lambda i: (0, 0)),
        out_shape=jax.ShapeDtypeStruct((_NP, 8 * _H), jnp.float32),
    )(jnp.reshape(eps, (1, 1)), y128, pp[0], pp[1], b1p, w2p, b2p)

    return out128.reshape(_N, _H)
